# baseline (speedup 1.0000x reference)
import functools

import jax
import jax.numpy as jnp
from jax import lax
from jax.experimental import pallas as pl
from jax.experimental.pallas import tpu as pltpu
from jax.experimental.pallas import tpu_sc as plsc

_F = 26
_B = 4096
_V = 100000
_D = 106
_NC = 2
_NS = 16
_L = 16
_NW = _NC * _NS
_BPW = _B // _NW


def _fields_gather_body(ids_hbm, tab_hbm, out_hbm, idx_v, rows_v, sem0, sem1):
    wid = lax.axis_index("s") * _NC + lax.axis_index("c")
    b0 = wid * _BPW
    pltpu.sync_copy(ids_hbm.at[:, pl.ds(b0, _BPW)], idx_v)
    for f in range(_F):
        for c in range(_BPW // _L):
            sl = (f, pl.ds(c * _L, _L))
            idx_v[sl] = idx_v[sl] + f * _V
    sems = (sem0, sem1)

    def start(f):
        return pltpu.async_copy(tab_hbm.at[idx_v.at[f]], rows_v.at[f % 2],
                                sems[f % 2])

    cp = start(0)
    for f in range(_F):
        nxt = start(f + 1) if f + 1 < _F else None
        cp.wait()
        pltpu.sync_copy(rows_v.at[f % 2], out_hbm.at[pl.ds(b0, _BPW), f])
        cp = nxt


@jax.jit
def kernel(ids, tables):
    tab_flat = tables.reshape(_F * _V, _D)
    run = pl.kernel(
        _fields_gather_body,
        out_type=jax.ShapeDtypeStruct((_B, _F, _D), jnp.float32),
        mesh=plsc.VectorSubcoreMesh(core_axis_name="c", subcore_axis_name="s"),
        compiler_params=pltpu.CompilerParams(use_tc_tiling_on_sc=False),
        scratch_types=[
            pltpu.VMEM((_F, _BPW), jnp.int32),
            pltpu.VMEM((2, _BPW, _D), jnp.float32),
            pltpu.SemaphoreType.DMA,
            pltpu.SemaphoreType.DMA,
        ],
    )
    out3 = run(ids, tab_flat)
    return out3.reshape(_B, _F * _D)


# speedup vs baseline: 1.2781x; 1.2781x over previous
import jax
import jax.numpy as jnp
from jax import lax
from jax.experimental import pallas as pl
from jax.experimental.pallas import tpu as pltpu
from jax.experimental.pallas import tpu_sc as plsc

_F = 26
_B = 4096
_V = 100000
_D = 106
_DP = 128
_NC = 2
_NS = 16
_L = 16
_NW = _NC * _NS
_BPW = _B // _NW


def _fields_gather_body(ids_hbm, tab_hbm, out_hbm, idx_v, rows_v, sem0, sem1):
    wid = lax.axis_index("s") * _NC + lax.axis_index("c")
    b0 = wid * _BPW
    pltpu.sync_copy(ids_hbm.at[:, pl.ds(b0, _BPW)], idx_v)
    for f in range(_F):
        for c in range(_BPW // _L):
            sl = (f, pl.ds(c * _L, _L))
            idx_v[sl] = idx_v[sl] + f * _V
    sems = (sem0, sem1)

    def start(f):
        return pltpu.async_copy(tab_hbm.at[idx_v.at[f]], rows_v.at[f % 2],
                                sems[f % 2])

    cp = start(0)
    for f in range(_F):
        nxt = start(f + 1) if f + 1 < _F else None
        cp.wait()
        pltpu.sync_copy(rows_v.at[f % 2],
                        out_hbm.at[pl.ds(b0, _BPW), pl.ds(f * _DP, _DP)])
        cp = nxt


@jax.jit
def kernel(ids, tables):
    tab_pad = jnp.pad(tables.reshape(_F * _V, _D), ((0, 0), (0, _DP - _D)))
    run = pl.kernel(
        _fields_gather_body,
        out_type=jax.ShapeDtypeStruct((_B, _F * _DP), jnp.float32),
        mesh=plsc.VectorSubcoreMesh(core_axis_name="c", subcore_axis_name="s"),
        compiler_params=pltpu.CompilerParams(use_tc_tiling_on_sc=False),
        scratch_types=[
            pltpu.VMEM((_F, _BPW), jnp.int32),
            pltpu.VMEM((2, _BPW, _DP), jnp.float32),
            pltpu.SemaphoreType.DMA,
            pltpu.SemaphoreType.DMA,
        ],
    )
    out = run(ids, tab_pad)
    return out.reshape(_B, _F, _DP)[:, :, :_D].reshape(_B, _F * _D)


# speedup vs baseline: 4.5356x; 3.5488x over previous
import jax
import jax.numpy as jnp
from jax import lax
from jax.experimental import pallas as pl
from jax.experimental.pallas import tpu as pltpu
from jax.experimental.pallas import tpu_sc as plsc

_F = 26
_B = 4096
_V = 100000
_D = 106
_DP = 128
_NC = 2
_NS = 16
_L = 16
_NW = _NC * _NS
_BPW = _B // _NW
_VBLK = 2048


def _transpose_pad_body(in_ref, out_ref):
    x = in_ref[0]
    xt = jnp.transpose(x, (1, 0))
    out_ref[0] = jnp.pad(xt, ((0, 0), (0, _DP - _D)))


def _fields_gather_body(ids_hbm, tab_hbm, out_hbm, idx_v, rows_v, sem0, sem1):
    wid = lax.axis_index("s") * _NC + lax.axis_index("c")
    b0 = wid * _BPW
    pltpu.sync_copy(ids_hbm.at[:, pl.ds(b0, _BPW)], idx_v)
    for f in range(_F):
        for c in range(_BPW // _L):
            sl = (f, pl.ds(c * _L, _L))
            idx_v[sl] = idx_v[sl] + f * _V
    sems = (sem0, sem1)

    def start(f):
        return pltpu.async_copy(tab_hbm.at[idx_v.at[f]], rows_v.at[f % 2],
                                sems[f % 2])

    cp = start(0)
    for f in range(_F):
        nxt = start(f + 1) if f + 1 < _F else None
        cp.wait()
        pltpu.sync_copy(rows_v.at[f % 2],
                        out_hbm.at[pl.ds(b0, _BPW), pl.ds(f * _DP, _DP)])
        cp = nxt


@jax.jit
def kernel(ids, tables):
    tab_t = jnp.transpose(tables, (0, 2, 1))
    tab128 = pl.pallas_call(
        _transpose_pad_body,
        grid=(_F, (_V + _VBLK - 1) // _VBLK),
        in_specs=[pl.BlockSpec((1, _D, _VBLK), lambda f, j: (f, 0, j))],
        out_specs=pl.BlockSpec((1, _VBLK, _DP), lambda f, j: (f, j, 0)),
        out_shape=jax.ShapeDtypeStruct((_F, _V, _DP), jnp.float32),
    )(tab_t)
    run = pl.kernel(
        _fields_gather_body,
        out_type=jax.ShapeDtypeStruct((_B, _F * _DP), jnp.float32),
        mesh=plsc.VectorSubcoreMesh(core_axis_name="c", subcore_axis_name="s"),
        compiler_params=pltpu.CompilerParams(use_tc_tiling_on_sc=False),
        scratch_types=[
            pltpu.VMEM((_F, _BPW), jnp.int32),
            pltpu.VMEM((2, _BPW, _DP), jnp.float32),
            pltpu.SemaphoreType.DMA,
            pltpu.SemaphoreType.DMA,
        ],
    )
    out = run(ids, tab128.reshape(_F * _V, _DP))
    return out.reshape(_B, _F, _DP)[:, :, :_D].reshape(_B, _F * _D)


# speedup vs baseline: 6.6008x; 1.4553x over previous
import jax
import jax.numpy as jnp
from jax import lax
from jax.experimental import pallas as pl
from jax.experimental.pallas import tpu as pltpu
from jax.experimental.pallas import tpu_sc as plsc

_F = 26
_B = 4096
_V = 100000
_D = 106
_DP = 128
_NC = 2
_NS = 16
_L = 16
_NW = _NC * _NS
_BPW = _B // _NW
_VBLK = 8192


def _transpose_pad_body(in_ref, out_ref):
    x = in_ref[0]
    xt = jnp.transpose(x, (1, 0))
    out_ref[0] = jnp.pad(xt, ((0, 0), (0, _DP - _D)))


def _fields_gather_body(ids_hbm, tab_hbm, out_hbm, idx_v, rows_v, sem0, sem1):
    wid = lax.axis_index("s") * _NC + lax.axis_index("c")
    b0 = wid * _BPW
    pltpu.sync_copy(ids_hbm.at[:, pl.ds(b0, _BPW)], idx_v)
    for f in range(_F):
        for c in range(_BPW // _L):
            sl = (f, pl.ds(c * _L, _L))
            idx_v[sl] = idx_v[sl] + f * _V
    sems = (sem0, sem1)

    def start(f):
        return pltpu.async_copy(tab_hbm.at[idx_v.at[f]], rows_v.at[f % 2],
                                sems[f % 2])

    cp = start(0)
    for f in range(_F):
        nxt = start(f + 1) if f + 1 < _F else None
        cp.wait()
        pltpu.sync_copy(rows_v.at[f % 2],
                        out_hbm.at[pl.ds(b0, _BPW), pl.ds(f * _DP, _DP)])
        cp = nxt


@jax.jit
def kernel(ids, tables):
    tab_t = jnp.transpose(tables, (0, 2, 1))
    tab128 = pl.pallas_call(
        _transpose_pad_body,
        grid=(_F, (_V + _VBLK - 1) // _VBLK),
        in_specs=[pl.BlockSpec((1, _D, _VBLK), lambda f, j: (f, 0, j))],
        out_specs=pl.BlockSpec((1, _VBLK, _DP), lambda f, j: (f, j, 0)),
        out_shape=jax.ShapeDtypeStruct((_F, _V, _DP), jnp.float32),
    )(tab_t)
    run = pl.kernel(
        _fields_gather_body,
        out_type=jax.ShapeDtypeStruct((_B, _F * _DP), jnp.float32),
        mesh=plsc.VectorSubcoreMesh(core_axis_name="c", subcore_axis_name="s"),
        compiler_params=pltpu.CompilerParams(use_tc_tiling_on_sc=False),
        scratch_types=[
            pltpu.VMEM((_F, _BPW), jnp.int32),
            pltpu.VMEM((2, _BPW, _DP), jnp.float32),
            pltpu.SemaphoreType.DMA,
            pltpu.SemaphoreType.DMA,
        ],
    )
    out = run(ids, tab128.reshape(_F * _V, _DP))
    return out.reshape(_B, _F, _DP)[:, :, :_D].reshape(_B, _F * _D)


# speedup vs baseline: 6.9889x; 1.0588x over previous
import jax
import jax.numpy as jnp
from jax import lax
from jax.experimental import pallas as pl
from jax.experimental.pallas import tpu as pltpu
from jax.experimental.pallas import tpu_sc as plsc

_F = 26
_B = 4096
_V = 100000
_D = 106
_DP = 128
_NC = 2
_NS = 16
_L = 16
_NW = _NC * _NS
_BPW = _B // _NW
_VBLK = 12800


def _transpose_pad_body(in_ref, out_ref):
    x = in_ref[0]
    xt = jnp.transpose(x, (1, 0))
    out_ref[0] = jnp.pad(xt, ((0, 0), (0, _DP - _D)))


def _fields_gather_body(ids_hbm, tab_hbm, out_hbm, idx_v, rows_v, sem0, sem1):
    wid = lax.axis_index("s") * _NC + lax.axis_index("c")
    b0 = wid * _BPW
    pltpu.sync_copy(ids_hbm.at[:, pl.ds(b0, _BPW)], idx_v)
    for f in range(_F):
        for c in range(_BPW // _L):
            sl = (f, pl.ds(c * _L, _L))
            idx_v[sl] = idx_v[sl] + f * _V
    sems = (sem0, sem1)

    def start(f):
        return pltpu.async_copy(tab_hbm.at[idx_v.at[f]], rows_v.at[f % 2],
                                sems[f % 2])

    cp = start(0)
    for f in range(_F):
        nxt = start(f + 1) if f + 1 < _F else None
        cp.wait()
        pltpu.sync_copy(rows_v.at[f % 2],
                        out_hbm.at[pl.ds(b0, _BPW), pl.ds(f * _DP, _DP)])
        cp = nxt


@jax.jit
def kernel(ids, tables):
    tab_t = jnp.transpose(tables, (0, 2, 1))
    tab128 = pl.pallas_call(
        _transpose_pad_body,
        grid=(_F, (_V + _VBLK - 1) // _VBLK),
        in_specs=[pl.BlockSpec((1, _D, _VBLK), lambda f, j: (f, 0, j))],
        out_specs=pl.BlockSpec((1, _VBLK, _DP), lambda f, j: (f, j, 0)),
        out_shape=jax.ShapeDtypeStruct((_F, _V, _DP), jnp.float32),
    )(tab_t)
    run = pl.kernel(
        _fields_gather_body,
        out_type=jax.ShapeDtypeStruct((_B, _F * _DP), jnp.float32),
        mesh=plsc.VectorSubcoreMesh(core_axis_name="c", subcore_axis_name="s"),
        compiler_params=pltpu.CompilerParams(use_tc_tiling_on_sc=False),
        scratch_types=[
            pltpu.VMEM((_F, _BPW), jnp.int32),
            pltpu.VMEM((2, _BPW, _DP), jnp.float32),
            pltpu.SemaphoreType.DMA,
            pltpu.SemaphoreType.DMA,
        ],
    )
    out = run(ids, tab128.reshape(_F * _V, _DP))
    return out.reshape(_B, _F, _DP)[:, :, :_D].reshape(_B, _F * _D)


# speedup vs baseline: 7.0573x; 1.0098x over previous
import jax
import jax.numpy as jnp
from jax import lax
from jax.experimental import pallas as pl
from jax.experimental.pallas import tpu as pltpu
from jax.experimental.pallas import tpu_sc as plsc

_F = 26
_B = 4096
_V = 100000
_D = 106
_DP = 128
_NC = 2
_NS = 16
_L = 16
_NW = _NC * _NS
_BPW = _B // _NW
_VBLK = 25600


def _transpose_pad_body(in_ref, out_ref):
    x = in_ref[0]
    xt = jnp.transpose(x, (1, 0))
    out_ref[0] = jnp.pad(xt, ((0, 0), (0, _DP - _D)))


def _fields_gather_body(ids_hbm, tab_hbm, out_hbm, idx_v, rows_v, sem0, sem1):
    wid = lax.axis_index("s") * _NC + lax.axis_index("c")
    b0 = wid * _BPW
    pltpu.sync_copy(ids_hbm.at[:, pl.ds(b0, _BPW)], idx_v)
    for f in range(_F):
        for c in range(_BPW // _L):
            sl = (f, pl.ds(c * _L, _L))
            idx_v[sl] = idx_v[sl] + f * _V
    sems = (sem0, sem1)

    def start(f):
        return pltpu.async_copy(tab_hbm.at[idx_v.at[f]], rows_v.at[f % 2],
                                sems[f % 2])

    cp = start(0)
    for f in range(_F):
        nxt = start(f + 1) if f + 1 < _F else None
        cp.wait()
        pltpu.sync_copy(rows_v.at[f % 2],
                        out_hbm.at[pl.ds(b0, _BPW), pl.ds(f * _DP, _DP)])
        cp = nxt


@jax.jit
def kernel(ids, tables):
    tab_t = jnp.transpose(tables, (0, 2, 1))
    tab128 = pl.pallas_call(
        _transpose_pad_body,
        grid=(_F, (_V + _VBLK - 1) // _VBLK),
        in_specs=[pl.BlockSpec((1, _D, _VBLK), lambda f, j: (f, 0, j))],
        out_specs=pl.BlockSpec((1, _VBLK, _DP), lambda f, j: (f, j, 0)),
        out_shape=jax.ShapeDtypeStruct((_F, _V, _DP), jnp.float32),
    )(tab_t)
    run = pl.kernel(
        _fields_gather_body,
        out_type=jax.ShapeDtypeStruct((_B, _F * _DP), jnp.float32),
        mesh=plsc.VectorSubcoreMesh(core_axis_name="c", subcore_axis_name="s"),
        compiler_params=pltpu.CompilerParams(use_tc_tiling_on_sc=False),
        scratch_types=[
            pltpu.VMEM((_F, _BPW), jnp.int32),
            pltpu.VMEM((2, _BPW, _DP), jnp.float32),
            pltpu.SemaphoreType.DMA,
            pltpu.SemaphoreType.DMA,
        ],
    )
    out = run(ids, tab128.reshape(_F * _V, _DP))
    return out.reshape(_B, _F, _DP)[:, :, :_D].reshape(_B, _F * _D)


# speedup vs baseline: 7.0610x; 1.0005x over previous
import jax
import jax.numpy as jnp
from jax import lax
from jax.experimental import pallas as pl
from jax.experimental.pallas import tpu as pltpu
from jax.experimental.pallas import tpu_sc as plsc

_F = 26
_B = 4096
_V = 100000
_D = 106
_DP = 128
_NC = 2
_NS = 16
_L = 16
_NW = _NC * _NS
_BPW = _B // _NW
_VBLK = 25600


def _transpose_pad_body(in_ref, out_ref):
    x = in_ref[0]
    out_ref[0, :, : _D] = jnp.transpose(x, (1, 0))


def _fields_gather_body(ids_hbm, tab_hbm, out_hbm, idx_v, rows_v, sem0, sem1):
    wid = lax.axis_index("s") * _NC + lax.axis_index("c")
    b0 = wid * _BPW
    pltpu.sync_copy(ids_hbm.at[:, pl.ds(b0, _BPW)], idx_v)
    for f in range(_F):
        for c in range(_BPW // _L):
            sl = (f, pl.ds(c * _L, _L))
            idx_v[sl] = idx_v[sl] + f * _V
    sems = (sem0, sem1)

    def start(f):
        return pltpu.async_copy(tab_hbm.at[idx_v.at[f]], rows_v.at[f % 2],
                                sems[f % 2])

    cp = start(0)
    for f in range(_F):
        nxt = start(f + 1) if f + 1 < _F else None
        cp.wait()
        pltpu.sync_copy(rows_v.at[f % 2],
                        out_hbm.at[pl.ds(b0, _BPW), pl.ds(f * _DP, _DP)])
        cp = nxt


@jax.jit
def kernel(ids, tables):
    tab_t = jnp.transpose(tables, (0, 2, 1))
    tab128 = pl.pallas_call(
        _transpose_pad_body,
        grid=(_F, (_V + _VBLK - 1) // _VBLK),
        in_specs=[pl.BlockSpec((1, _D, _VBLK), lambda f, j: (f, 0, j))],
        out_specs=pl.BlockSpec((1, _VBLK, _DP), lambda f, j: (f, j, 0)),
        out_shape=jax.ShapeDtypeStruct((_F, _V, _DP), jnp.float32),
    )(tab_t)
    run = pl.kernel(
        _fields_gather_body,
        out_type=jax.ShapeDtypeStruct((_B, _F * _DP), jnp.float32),
        mesh=plsc.VectorSubcoreMesh(core_axis_name="c", subcore_axis_name="s"),
        compiler_params=pltpu.CompilerParams(use_tc_tiling_on_sc=False),
        scratch_types=[
            pltpu.VMEM((_F, _BPW), jnp.int32),
            pltpu.VMEM((2, _BPW, _DP), jnp.float32),
            pltpu.SemaphoreType.DMA,
            pltpu.SemaphoreType.DMA,
        ],
    )
    out = run(ids, tab128.reshape(_F * _V, _DP))
    return out.reshape(_B, _F, _DP)[:, :, :_D].reshape(_B, _F * _D)
